# baseline (speedup 1.0000x reference)
import functools

import jax
import jax.numpy as jnp
from jax import lax
from jax.experimental import pallas as pl
from jax.experimental.pallas import tpu as pltpu
from jax.experimental.pallas import tpu_sc as plsc

_B = 16384
_J = 12
_V = 160
_D = 8
_NC = 2
_NS = 16
_NW = _NC * _NS
_BPW = _B // _NW
_RB = _BPW // 16


def _table_body(emb_ref, w3_ref, b3_ref, l_ref):
    e = jnp.maximum(emb_ref[...], 0.0)
    w = w3_ref[...]
    for j in range(_J):
        blk = w[:, _D * j:_D * (j + 1)]
        lj = lax.dot_general(blk, e, (((1,), (1,)), ((), ())),
                             preferred_element_type=jnp.float32)
        if j == 0:
            lj = lj + b3_ref[...]
        l_ref[:, _V * j:_V * (j + 1)] = lj


_build_table = pl.pallas_call(
    _table_body,
    out_shape=jax.ShapeDtypeStruct((3, _J * _V), jnp.float32),
)


@functools.partial(
    pl.kernel,
    out_type=jax.ShapeDtypeStruct((_B, 3), jnp.float32),
    mesh=plsc.VectorSubcoreMesh(core_axis_name="c", subcore_axis_name="s"),
    compiler_params=pltpu.CompilerParams(needs_layout_passes=False,
                                         use_tc_tiling_on_sc=False),
    scratch_types=[
        pltpu.VMEM((3, _J * _V), jnp.float32),
        pltpu.VMEM((_BPW, _J), jnp.int32),
        pltpu.VMEM((_BPW, 3), jnp.float32),
    ],
)
def _sc_lookup(l_hbm, idx_hbm, out_hbm, l_v, idx_v, out_v):
    wid = lax.axis_index("s") * _NC + lax.axis_index("c")
    base = wid * _BPW
    pltpu.sync_copy(l_hbm, l_v)

    lanes = jax.lax.iota(jnp.int32, 16)
    csplat = [jnp.full((16,), c, jnp.int32) for c in range(3)]

    def body(rb, carry):
        rows = lanes + rb * 16
        acc0 = jnp.zeros((16,), jnp.float32)
        acc1 = jnp.zeros((16,), jnp.float32)
        acc2 = jnp.zeros((16,), jnp.float32)
        for j in range(_J):
            vj = plsc.load_gather(idx_v, [rows, jnp.full((16,), j, jnp.int32)])
            fidx = vj + (j * _V)
            acc0 = acc0 + plsc.load_gather(l_v, [csplat[0], fidx])
            acc1 = acc1 + plsc.load_gather(l_v, [csplat[1], fidx])
            acc2 = acc2 + plsc.load_gather(l_v, [csplat[2], fidx])
        for c, acc in enumerate((acc0, acc1, acc2)):
            plsc.store_scatter(out_v, [rows, jnp.full((16,), c, jnp.int32)], acc)
        return carry

    pltpu.sync_copy(idx_hbm.at[pl.ds(base, _BPW), :], idx_v)
    lax.fori_loop(0, _RB, body, 0)
    pltpu.sync_copy(out_v, out_hbm.at[pl.ds(base, _BPW), :])


def kernel(pose_indices, image, emb_table, W3, b3):
    del image
    l_table = _build_table(emb_table, W3, b3.reshape(3, 1))
    return _sc_lookup(l_table, pose_indices)


# speedup vs baseline: 2.2253x; 2.2253x over previous
import functools

import jax
import jax.numpy as jnp
from jax import lax
from jax.experimental import pallas as pl
from jax.experimental.pallas import tpu as pltpu
from jax.experimental.pallas import tpu_sc as plsc

_B = 16384
_J = 12
_V = 160
_D = 8
_NC = 2
_NS = 16
_NW = _NC * _NS
_BPW = _B // _NW
_RB = _BPW // 16


def _table_body(emb_ref, w3_ref, b3_ref, l_ref):
    e = jnp.maximum(emb_ref[...], 0.0)
    w = w3_ref[...]
    for j in range(_J):
        blk = w[:, _D * j:_D * (j + 1)]
        lj = lax.dot_general(blk, e, (((1,), (1,)), ((), ())),
                             preferred_element_type=jnp.float32)
        if j == 0:
            lj = lj + b3_ref[...]
        l_ref[:, _V * j:_V * (j + 1)] = lj


_build_table = pl.pallas_call(
    _table_body,
    out_shape=jax.ShapeDtypeStruct((3, _J * _V), jnp.float32),
)


@functools.partial(
    pl.kernel,
    out_type=jax.ShapeDtypeStruct((3, _B), jnp.float32),
    mesh=plsc.VectorSubcoreMesh(core_axis_name="c", subcore_axis_name="s"),
    compiler_params=pltpu.CompilerParams(needs_layout_passes=False),
    scratch_types=[
        pltpu.VMEM((3, _J * _V), jnp.float32),
        pltpu.VMEM((_J, _BPW), jnp.int32),
        pltpu.VMEM((3, _BPW), jnp.float32),
    ],
)
def _sc_lookup(l_hbm, idxt_hbm, outt_hbm, l_v, idx_v, out_v):
    wid = lax.axis_index("s") * _NC + lax.axis_index("c")
    base = wid * _BPW
    pltpu.sync_copy(idxt_hbm.at[:, pl.ds(base, _BPW)], idx_v)
    pltpu.sync_copy(l_hbm, l_v)

    lanes = jax.lax.iota(jnp.int32, 16)
    csplat = [jnp.full((16,), c, jnp.int32) for c in range(3)]
    jsplat = [jnp.full((16,), j, jnp.int32) for j in range(_J)]

    def body(rb, carry):
        rows = lanes + rb * 16
        acc0 = jnp.zeros((16,), jnp.float32)
        acc1 = jnp.zeros((16,), jnp.float32)
        acc2 = jnp.zeros((16,), jnp.float32)
        for j in range(_J):
            vj = plsc.load_gather(idx_v, [jsplat[j], rows])
            fidx = vj + (j * _V)
            acc0 = acc0 + plsc.load_gather(l_v, [csplat[0], fidx])
            acc1 = acc1 + plsc.load_gather(l_v, [csplat[1], fidx])
            acc2 = acc2 + plsc.load_gather(l_v, [csplat[2], fidx])
        for c, acc in enumerate((acc0, acc1, acc2)):
            plsc.store_scatter(out_v, [csplat[c], rows], acc)
        return carry

    lax.fori_loop(0, _RB, body, 0)
    pltpu.sync_copy(out_v, outt_hbm.at[:, pl.ds(base, _BPW)])


def kernel(pose_indices, image, emb_table, W3, b3):
    del image
    l_table = _build_table(emb_table, W3, b3.reshape(3, 1))
    out_t = _sc_lookup(l_table, pose_indices.T)
    return out_t.T


# speedup vs baseline: 2.2377x; 1.0055x over previous
import functools

import jax
import jax.numpy as jnp
from jax import lax
from jax.experimental import pallas as pl
from jax.experimental.pallas import tpu as pltpu
from jax.experimental.pallas import tpu_sc as plsc

_B = 16384
_J = 12
_V = 160
_D = 8
_NC = 2
_NS = 16
_NW = _NC * _NS
_BPW = _B // _NW
_RB = _BPW // 16


def _table_body(emb_ref, w3_ref, b3_ref, l_ref):
    e = jnp.maximum(emb_ref[...], 0.0)
    w = w3_ref[...]
    for j in range(_J):
        blk = w[:, _D * j:_D * (j + 1)]
        lj = lax.dot_general(blk, e, (((1,), (1,)), ((), ())),
                             preferred_element_type=jnp.float32)
        if j == 0:
            lj = lj + b3_ref[...]
        l_ref[:, _V * j:_V * (j + 1)] = lj


_build_table = pl.pallas_call(
    _table_body,
    out_shape=jax.ShapeDtypeStruct((3, _J * _V), jnp.float32),
)


@functools.partial(
    pl.kernel,
    out_type=jax.ShapeDtypeStruct((3, _B), jnp.float32),
    mesh=plsc.VectorSubcoreMesh(core_axis_name="c", subcore_axis_name="s"),
    compiler_params=pltpu.CompilerParams(needs_layout_passes=False),
    scratch_types=[
        pltpu.VMEM((3, _J * _V), jnp.float32),
        pltpu.VMEM((_J, _BPW), jnp.int32),
        pltpu.VMEM((3, _BPW), jnp.float32),
    ],
)
def _sc_lookup(l_hbm, idxt_hbm, outt_hbm, l_v, idx_v, out_v):
    wid = lax.axis_index("s") * _NC + lax.axis_index("c")
    base = wid * _BPW
    pltpu.sync_copy(idxt_hbm.at[:, pl.ds(base, _BPW)], idx_v)
    pltpu.sync_copy(l_hbm, l_v)

    csplat = [jnp.full((16,), c, jnp.int32) for c in range(3)]

    @plsc.parallel_loop(0, _RB, 1, unroll=2)
    def body(rb):
        off = rb * 16
        acc0 = jnp.zeros((16,), jnp.float32)
        acc1 = jnp.zeros((16,), jnp.float32)
        acc2 = jnp.zeros((16,), jnp.float32)
        for j in range(_J):
            vj = idx_v[j, pl.ds(off, 16)]
            fidx = vj + (j * _V)
            acc0 = acc0 + plsc.load_gather(l_v, [csplat[0], fidx])
            acc1 = acc1 + plsc.load_gather(l_v, [csplat[1], fidx])
            acc2 = acc2 + plsc.load_gather(l_v, [csplat[2], fidx])
        out_v[0, pl.ds(off, 16)] = acc0
        out_v[1, pl.ds(off, 16)] = acc1
        out_v[2, pl.ds(off, 16)] = acc2
    pltpu.sync_copy(out_v, outt_hbm.at[:, pl.ds(base, _BPW)])


def kernel(pose_indices, image, emb_table, W3, b3):
    del image
    l_table = _build_table(emb_table, W3, b3.reshape(3, 1))
    out_t = _sc_lookup(l_table, pose_indices.T)
    return out_t.T
